# Initial kernel scaffold; baseline (speedup 1.0000x reference)
import jax
import jax.numpy as jnp
from jax.experimental import pallas as pl


def kernel(x, W_real, W_imag):
    raise NotImplementedError("write your pallas kernel here")


# speedup vs baseline: 1.5257x; 1.5257x over previous
import functools

import jax
import jax.numpy as jnp
from jax import lax
from jax.experimental import pallas as pl
from jax.experimental.pallas import tpu as pltpu
from jax.experimental.pallas import tpu_sc as plsc

_VOCAB = 100000
_DIM = 128
_B = 4096
_L = 50
_N = _B * _L
_NW = 32
_PER_W = _N // _NW
_CHUNK = 128
_NCHUNK = _PER_W // _CHUNK

_mesh = plsc.VectorSubcoreMesh(core_axis_name="c", subcore_axis_name="s")


@functools.partial(
    pl.kernel,
    mesh=_mesh,
    out_type=(
        jax.ShapeDtypeStruct((_N, _DIM), jnp.float32),
        jax.ShapeDtypeStruct((_N, _DIM), jnp.float32),
    ),
    scratch_types=[
        pltpu.VMEM((_PER_W,), jnp.int32),
        pltpu.VMEM((_CHUNK, _DIM), jnp.float32),
        pltpu.VMEM((_CHUNK, _DIM), jnp.float32),
        pltpu.SemaphoreType.DMA,
    ],
)
def _gather2(x_hbm, wr_hbm, wi_hbm, outr_hbm, outi_hbm, idx_v, bufr, bufi, sem):
    wid = lax.axis_index("s") * 2 + lax.axis_index("c")
    base = wid * _PER_W
    pltpu.sync_copy(x_hbm.at[pl.ds(base, _PER_W)], idx_v)

    def body(i, carry):
        off = i * _CHUNK
        idx = idx_v.at[pl.ds(off, _CHUNK)]
        pltpu.async_copy(wr_hbm.at[idx], bufr, sem).wait()
        pltpu.async_copy(wi_hbm.at[idx], bufi, sem).wait()
        pltpu.sync_copy(bufr, outr_hbm.at[pl.ds(base + off, _CHUNK)])
        pltpu.sync_copy(bufi, outi_hbm.at[pl.ds(base + off, _CHUNK)])
        return carry

    lax.fori_loop(0, _NCHUNK, body, 0)


@jax.jit
def kernel(x, W_real, W_imag):
    real, imag = _gather2(x.reshape(_N), W_real, W_imag)
    out = lax.complex(real, imag)
    return out.reshape(_B, _L, _DIM)


# speedup vs baseline: 1.5722x; 1.0305x over previous
import functools

import jax
import jax.numpy as jnp
from jax import lax
from jax.experimental import pallas as pl
from jax.experimental.pallas import tpu as pltpu
from jax.experimental.pallas import tpu_sc as plsc

_VOCAB = 100000
_DIM = 128
_B = 4096
_L = 50
_N = _B * _L
_NW = 32
_PER_W = _N // _NW
_CHUNK = 64
_NCHUNK = _PER_W // _CHUNK
_NBUF = 4
_ROUNDS = _NCHUNK // _NBUF - 1

_mesh = plsc.VectorSubcoreMesh(core_axis_name="c", subcore_axis_name="s")


@functools.partial(
    pl.kernel,
    mesh=_mesh,
    out_type=(
        jax.ShapeDtypeStruct((_N, _DIM), jnp.float32),
        jax.ShapeDtypeStruct((_N, _DIM), jnp.float32),
    ),
    scratch_types=[
        pltpu.VMEM((_PER_W,), jnp.int32),
        pltpu.VMEM((_NBUF, _CHUNK, _DIM), jnp.float32),
        pltpu.VMEM((_NBUF, _CHUNK, _DIM), jnp.float32),
        pltpu.SemaphoreType.DMA((_NBUF,)),
        pltpu.SemaphoreType.DMA((_NBUF,)),
    ],
)
def _gather2(x_hbm, wr_hbm, wi_hbm, outr_hbm, outi_hbm,
             idx_v, bufr, bufi, sem_in, sem_out):
    wid = lax.axis_index("s") * 2 + lax.axis_index("c")
    base = wid * _PER_W
    pltpu.sync_copy(x_hbm.at[pl.ds(base, _PER_W)], idx_v)

    def start_gather(c, b):
        idx = idx_v.at[pl.ds(c * _CHUNK, _CHUNK)]
        pltpu.async_copy(wr_hbm.at[idx], bufr.at[b], sem_in.at[b])
        pltpu.async_copy(wi_hbm.at[idx], bufi.at[b], sem_in.at[b])

    def wait_gather(b):
        pltpu.make_async_copy(wr_hbm.at[pl.ds(0, _CHUNK)], bufr.at[b],
                              sem_in.at[b]).wait()
        pltpu.make_async_copy(wi_hbm.at[pl.ds(0, _CHUNK)], bufi.at[b],
                              sem_in.at[b]).wait()

    def start_out(c, b):
        dst = pl.ds(base + c * _CHUNK, _CHUNK)
        pltpu.async_copy(bufr.at[b], outr_hbm.at[dst], sem_out.at[b])
        pltpu.async_copy(bufi.at[b], outi_hbm.at[dst], sem_out.at[b])

    def wait_out(c, b):
        dst = pl.ds(base + c * _CHUNK, _CHUNK)
        pltpu.make_async_copy(bufr.at[b], outr_hbm.at[dst],
                              sem_out.at[b]).wait()
        pltpu.make_async_copy(bufi.at[b], outi_hbm.at[dst],
                              sem_out.at[b]).wait()

    for b in range(_NBUF):
        start_gather(b, b)

    def round_body(g, carry):
        for b in range(_NBUF):
            c = g * _NBUF + b
            wait_gather(b)
            start_out(c, b)
            wait_out(c, b)
            start_gather(c + _NBUF, b)
        return carry

    lax.fori_loop(0, _ROUNDS, round_body, 0)

    for b in range(_NBUF):
        c = _ROUNDS * _NBUF + b
        wait_gather(b)
        start_out(c, b)
        wait_out(c, b)


@jax.jit
def kernel(x, W_real, W_imag):
    real, imag = _gather2(x.reshape(_N), W_real, W_imag)
    out = lax.complex(real, imag)
    return out.reshape(_B, _L, _DIM)


# speedup vs baseline: 2.0873x; 1.3276x over previous
import functools

import jax
import jax.numpy as jnp
from jax import lax
from jax.experimental import pallas as pl
from jax.experimental.pallas import tpu as pltpu
from jax.experimental.pallas import tpu_sc as plsc

_VOCAB = 100000
_DIM = 128
_B = 4096
_L = 50
_N = _B * _L
_NW = 32
_PER_W = _N // _NW
_CHUNK = 64
_NCHUNK = _PER_W // _CHUNK
_NBUF = 4
_ROUNDS = _NCHUNK // _NBUF - 1

_mesh = plsc.VectorSubcoreMesh(core_axis_name="c", subcore_axis_name="s")


@functools.partial(
    pl.kernel,
    mesh=_mesh,
    out_type=(
        jax.ShapeDtypeStruct((_N, _DIM), jnp.float32),
        jax.ShapeDtypeStruct((_N, _DIM), jnp.float32),
    ),
    scratch_types=[
        pltpu.VMEM((_PER_W,), jnp.int32),
        pltpu.VMEM((_NBUF, _CHUNK, _DIM), jnp.float32),
        pltpu.VMEM((_NBUF, _CHUNK, _DIM), jnp.float32),
        pltpu.SemaphoreType.DMA((_NBUF,)),
        pltpu.SemaphoreType.DMA((_NBUF,)),
    ],
)
def _gather2(x_hbm, wr_hbm, wi_hbm, outr_hbm, outi_hbm,
             idx_v, bufr, bufi, sem_in, sem_out):
    wid = lax.axis_index("s") * 2 + lax.axis_index("c")
    base = wid * _PER_W
    pltpu.sync_copy(x_hbm.at[pl.ds(base, _PER_W)], idx_v)

    def start_gather(c, b):
        idx = idx_v.at[pl.ds(c * _CHUNK, _CHUNK)]
        pltpu.async_copy(wr_hbm.at[idx], bufr.at[b], sem_in.at[b])
        pltpu.async_copy(wi_hbm.at[idx], bufi.at[b], sem_in.at[b])

    def wait_gather(b):
        pltpu.make_async_copy(wr_hbm.at[pl.ds(0, _CHUNK)], bufr.at[b],
                              sem_in.at[b]).wait()
        pltpu.make_async_copy(wi_hbm.at[pl.ds(0, _CHUNK)], bufi.at[b],
                              sem_in.at[b]).wait()

    def start_out(c, b):
        dst = pl.ds(base + c * _CHUNK, _CHUNK)
        pltpu.async_copy(bufr.at[b], outr_hbm.at[dst], sem_out.at[b])
        pltpu.async_copy(bufi.at[b], outi_hbm.at[dst], sem_out.at[b])

    def wait_out(c, b):
        dst = pl.ds(base + c * _CHUNK, _CHUNK)
        pltpu.make_async_copy(bufr.at[b], outr_hbm.at[dst],
                              sem_out.at[b]).wait()
        pltpu.make_async_copy(bufi.at[b], outi_hbm.at[dst],
                              sem_out.at[b]).wait()

    for b in range(_NBUF):
        start_gather(b, b)

    def round_body(g, carry):
        for b in range(_NBUF):
            c = g * _NBUF + b
            wait_gather(b)
            start_out(c, b)
            wait_out(c, b)
            start_gather(c + _NBUF, b)
        return carry

    lax.fori_loop(0, _ROUNDS, round_body, 0)

    for b in range(_NBUF):
        c = _ROUNDS * _NBUF + b
        wait_gather(b)
        start_out(c, b)
        wait_out(c, b)


@jax.jit
def kernel(x, W_real, W_imag):
    xt = x.T.reshape(_N)
    real, imag = _gather2(xt, W_real, W_imag)
    real3 = real.reshape(_L, _B, _DIM).transpose(1, 0, 2)
    imag3 = imag.reshape(_L, _B, _DIM).transpose(1, 0, 2)
    return lax.complex(real3, imag3)
